# Initial kernel scaffold; baseline (speedup 1.0000x reference)
import jax
import jax.numpy as jnp
from jax.experimental import pallas as pl


def kernel(X, ind, W):
    raise NotImplementedError("write your pallas kernel here")


# speedup vs baseline: 5627.6656x; 5627.6656x over previous
import functools

import jax
import jax.numpy as jnp
from jax import lax
from jax.experimental import pallas as pl
from jax.experimental.pallas import tpu as pltpu
from jax.experimental.pallas import tpu_sc as plsc

_B, _T, _I = 2, 2048, 1024
_E, _K, _J = 8, 512, 512

_INFO = plsc.get_sparse_core_info()
_NC, _NS = _INFO.num_cores, _INFO.num_subcores
_NW = _NC * _NS

_ROWS = _B * _E * _K
_RPW = _ROWS // _NW
_CHUNK = 64
_NCHUNK = _RPW // _CHUNK
_WPB = (_E * _K) // _RPW


def _sc_gather(x_flat, ind_flat):
    mesh = plsc.VectorSubcoreMesh(core_axis_name="c", subcore_axis_name="s")

    @functools.partial(
        pl.kernel,
        mesh=mesh,
        out_type=jax.ShapeDtypeStruct((_ROWS, _I), jnp.float32),
        scratch_types=[
            pltpu.VMEM((_CHUNK,), jnp.int32),
            pltpu.VMEM((_CHUNK, _I), jnp.float32),
            pltpu.SemaphoreType.DMA,
        ],
    )
    def gather_kernel(x_hbm, ind_hbm, out_hbm, idx_v, rows_v, sem):
        wid = lax.axis_index("s") * _NC + lax.axis_index("c")
        base = wid * _RPW
        boff = (wid // _WPB) * _T

        def chunk_body(c, carry):
            cbase = base + c * _CHUNK
            pltpu.sync_copy(ind_hbm.at[pl.ds(cbase, _CHUNK)], idx_v)
            for i in range(_CHUNK // 16):
                sl = pl.ds(i * 16, 16)
                idx_v[sl] = idx_v[sl] + boff
            pltpu.async_copy(x_hbm.at[idx_v], rows_v, sem).wait()
            pltpu.sync_copy(rows_v, out_hbm.at[pl.ds(cbase, _CHUNK)])
            return carry

        lax.fori_loop(0, _NCHUNK, chunk_body, 0)

    return gather_kernel(x_flat, ind_flat)


def _tc_matmul(xg, w):
    def mm_kernel(x_ref, w_ref, o_ref):
        o_ref[0] = jnp.dot(x_ref[0], w_ref[0],
                           preferred_element_type=jnp.float32)

    return pl.pallas_call(
        mm_kernel,
        grid=(_E, _B),
        in_specs=[
            pl.BlockSpec((1, _K, _I), lambda e, b: (b * _E + e, 0, 0)),
            pl.BlockSpec((1, _I, _J), lambda e, b: (e, 0, 0)),
        ],
        out_specs=pl.BlockSpec((1, _K, _J), lambda e, b: (b * _E + e, 0, 0)),
        out_shape=jax.ShapeDtypeStruct((_B * _E, _K, _J), jnp.float32),
    )(xg, w)


def kernel(X, ind, W):
    x_flat = X.reshape(_B * _T, _I)
    ind_flat = ind.reshape(_ROWS)
    xg = _sc_gather(x_flat, ind_flat)
    y = _tc_matmul(xg.reshape(_B * _E, _K, _I), W)
    return y.reshape(_B, _E, _K, _J)
